# baseline (speedup 1.0000x reference)
import jax
import jax.numpy as jnp
from jax import lax
from jax.experimental import pallas as pl
from jax.experimental.pallas import tpu as pltpu
from jax.experimental.pallas import tpu_sc as plsc

_NB = 4


def _sc_gather(idx1d, weight, batch, fields, embed_dim):
    mesh = plsc.VectorSubcoreMesh(
        core_axis_name="core", subcore_axis_name="subcore"
    )
    info = plsc.get_sparse_core_info()
    nw = info.num_cores * info.num_subcores
    window = _NB * fields
    b_per_w = batch // nw
    steps = b_per_w // _NB
    idx_per_w = b_per_w * fields

    @pl.kernel(
        out_type=jax.ShapeDtypeStruct(
            (batch, fields, embed_dim), weight.dtype
        ),
        mesh=mesh,
        compiler_params=pltpu.CompilerParams(use_tc_tiling_on_sc=True),
        scratch_types=[
            pltpu.VMEM((idx_per_w,), jnp.int32),
            pltpu.VMEM((window, embed_dim), jnp.float32),
            pltpu.SemaphoreType.DMA,
        ],
    )
    def gather_kernel(x_hbm, i_hbm, o_hbm, idx_v, rows_v, sem):
        c = lax.axis_index("core")
        s = lax.axis_index("subcore")
        wid = s * info.num_cores + c
        pltpu.sync_copy(i_hbm.at[pl.ds(wid * idx_per_w, idx_per_w)], idx_v)
        b_base = wid * b_per_w

        @pl.loop(0, steps)
        def _(step):
            off = pl.multiple_of(step * window, 8)
            pltpu.sync_copy(
                x_hbm.at[idx_v.at[pl.ds(off, window)]], rows_v
            )
            pltpu.sync_copy(
                rows_v.reshape(_NB, fields, embed_dim),
                o_hbm.at[pl.ds(b_base + step * _NB, _NB)],
            )

    return gather_kernel(weight, idx1d)


def kernel(indices, weight):
    batch, fields = indices.shape
    vocab, embed_dim = weight.shape
    idx1d = indices.reshape(batch * fields).astype(jnp.int32)
    return _sc_gather(idx1d, weight, batch, fields, embed_dim)


# speedup vs baseline: 1.2681x; 1.2681x over previous
import jax
import jax.numpy as jnp
from jax import lax
from jax.experimental import pallas as pl
from jax.experimental.pallas import tpu as pltpu
from jax.experimental.pallas import tpu_sc as plsc

_NB = 4
_NBUF = 4


def _sc_gather(idx1d, weight, batch, fields, embed_dim):
    mesh = plsc.VectorSubcoreMesh(
        core_axis_name="core", subcore_axis_name="subcore"
    )
    info = plsc.get_sparse_core_info()
    nw = info.num_cores * info.num_subcores
    window = _NB * fields
    b_per_w = batch // nw
    steps = b_per_w // _NB
    groups = steps // _NBUF - 1
    idx_per_w = b_per_w * fields

    @pl.kernel(
        out_type=jax.ShapeDtypeStruct(
            (batch, fields, embed_dim), weight.dtype
        ),
        mesh=mesh,
        scratch_types=[
            pltpu.VMEM((idx_per_w,), jnp.int32),
            pltpu.VMEM((_NBUF, window, embed_dim), jnp.float32),
            pltpu.SemaphoreType.DMA((_NBUF,)),
            pltpu.SemaphoreType.DMA((_NBUF,)),
        ],
    )
    def gather_kernel(x_hbm, i_hbm, o_hbm, idx_v, rows_v, gsem, wsem):
        c = lax.axis_index("core")
        s = lax.axis_index("subcore")
        wid = s * info.num_cores + c
        pltpu.sync_copy(i_hbm.at[pl.ds(wid * idx_per_w, idx_per_w)], idx_v)
        b_base = wid * b_per_w

        def issue_gather(step, nb):
            off = pl.multiple_of(step * window, 8)
            pltpu.async_copy(
                x_hbm.at[idx_v.at[pl.ds(off, window)]],
                rows_v.at[nb],
                gsem.at[nb],
            )

        def wait_gather(nb):
            pltpu.make_async_copy(
                x_hbm.at[idx_v.at[pl.ds(0, window)]],
                rows_v.at[nb],
                gsem.at[nb],
            ).wait()

        def issue_write(step, nb):
            pltpu.async_copy(
                rows_v.at[nb].reshape(_NB, fields, embed_dim),
                o_hbm.at[pl.ds(b_base + step * _NB, _NB)],
                wsem.at[nb],
            )

        def wait_write(nb):
            pltpu.make_async_copy(
                rows_v.at[nb].reshape(_NB, fields, embed_dim),
                o_hbm.at[pl.ds(b_base, _NB)],
                wsem.at[nb],
            ).wait()

        for nb in range(_NBUF):
            issue_gather(nb, nb)

        @pl.loop(0, groups)
        def _(grp):
            base = grp * _NBUF
            for nb in range(_NBUF):
                wait_gather(nb)
                issue_write(base + nb, nb)
            for nb in range(_NBUF):
                wait_write(nb)
                issue_gather(base + _NBUF + nb, nb)

        base = groups * _NBUF
        for nb in range(_NBUF):
            wait_gather(nb)
            issue_write(base + nb, nb)
        for nb in range(_NBUF):
            wait_write(nb)

    return gather_kernel(weight, idx1d)


def kernel(indices, weight):
    batch, fields = indices.shape
    vocab, embed_dim = weight.shape
    idx1d = indices.reshape(batch * fields).astype(jnp.int32)
    return _sc_gather(idx1d, weight, batch, fields, embed_dim)
